# Initial kernel scaffold; baseline (speedup 1.0000x reference)
import jax
import jax.numpy as jnp
from jax.experimental import pallas as pl


def kernel(x, centroids):
    raise NotImplementedError("write your pallas kernel here")


# speedup vs baseline: 54.1836x; 54.1836x over previous
import jax
import jax.numpy as jnp
from jax.experimental import pallas as pl

BLOCK_N = 256


def _assign_kernel(x_ref, ct_ref, out_ref):
    x_blk = x_ref[...]
    ct = ct_ref[...]
    s = jnp.dot(x_blk, ct,
                precision=jax.lax.Precision.HIGHEST,
                preferred_element_type=jnp.float32)
    cn2 = jnp.sum(ct * ct, axis=0)
    r2 = cn2[None, :] - 2.0 * s
    out_ref[...] = jnp.argmin(r2, axis=1).astype(jnp.int32)


def kernel(x, centroids):
    n, d = x.shape
    k = centroids.shape[0]
    ct = centroids.T
    grid = (n // BLOCK_N,)
    assignments = pl.pallas_call(
        _assign_kernel,
        grid=grid,
        in_specs=[
            pl.BlockSpec((BLOCK_N, d), lambda i: (i, 0)),
            pl.BlockSpec((d, k), lambda i: (0, 0)),
        ],
        out_specs=pl.BlockSpec((BLOCK_N,), lambda i: (i,)),
        out_shape=jax.ShapeDtypeStruct((n,), jnp.int32),
    )(x, ct)
    return (centroids[None, :, :], assignments)


# speedup vs baseline: 62.1245x; 1.1466x over previous
import jax
import jax.numpy as jnp
from jax.experimental import pallas as pl

BLOCK_N = 512


def _assign_kernel(x_ref, ct_ref, out_ref):
    x_blk = x_ref[...]
    ct = ct_ref[...]
    s = jnp.dot(x_blk, ct,
                precision=jax.lax.Precision.HIGHEST,
                preferred_element_type=jnp.float32)
    cn2 = jnp.sum(ct * ct, axis=0)
    r2 = cn2[None, :] - 2.0 * s
    out_ref[...] = jnp.argmin(r2, axis=1).astype(jnp.int32)


def kernel(x, centroids):
    n, d = x.shape
    k = centroids.shape[0]
    ct = centroids.T
    grid = (n // BLOCK_N,)
    assignments = pl.pallas_call(
        _assign_kernel,
        grid=grid,
        in_specs=[
            pl.BlockSpec((BLOCK_N, d), lambda i: (i, 0)),
            pl.BlockSpec((d, k), lambda i: (0, 0)),
        ],
        out_specs=pl.BlockSpec((BLOCK_N,), lambda i: (i,)),
        out_shape=jax.ShapeDtypeStruct((n,), jnp.int32),
    )(x, ct)
    return (centroids[None, :, :], assignments)


# speedup vs baseline: 76.5062x; 1.2315x over previous
import jax
import jax.numpy as jnp
from jax.experimental import pallas as pl

BLOCK_N = 512


def _split_hi_lo(a):
    hi_f32 = jax.lax.reduce_precision(a, exponent_bits=8, mantissa_bits=7)
    return hi_f32.astype(jnp.bfloat16), (a - hi_f32).astype(jnp.bfloat16)


def _assign_kernel(xh_ref, xl_ref, cth_ref, ctl_ref, ct_ref, out_ref):
    xh = xh_ref[...]
    xl = xl_ref[...]
    cth = cth_ref[...]
    ctl = ctl_ref[...]
    dot = lambda a, b: jnp.dot(a, b, preferred_element_type=jnp.float32)
    s = dot(xh, cth) + (dot(xh, ctl) + dot(xl, cth))
    ct = ct_ref[...]
    cn2 = jnp.sum(ct * ct, axis=0)
    r2 = cn2[None, :] - 2.0 * s
    out_ref[...] = jnp.argmin(r2, axis=1).astype(jnp.int32)


def kernel(x, centroids):
    n, d = x.shape
    k = centroids.shape[0]
    ct = centroids.T
    cth, ctl = _split_hi_lo(ct)
    xh, xl = _split_hi_lo(x)
    grid = (n // BLOCK_N,)
    assignments = pl.pallas_call(
        _assign_kernel,
        grid=grid,
        in_specs=[
            pl.BlockSpec((BLOCK_N, d), lambda i: (i, 0)),
            pl.BlockSpec((BLOCK_N, d), lambda i: (i, 0)),
            pl.BlockSpec((d, k), lambda i: (0, 0)),
            pl.BlockSpec((d, k), lambda i: (0, 0)),
            pl.BlockSpec((d, k), lambda i: (0, 0)),
        ],
        out_specs=pl.BlockSpec((BLOCK_N,), lambda i: (i,)),
        out_shape=jax.ShapeDtypeStruct((n,), jnp.int32),
    )(xh, xl, cth, ctl, ct)
    return (centroids[None, :, :], assignments)
